# baseline (speedup 1.0000x reference)
import functools

import jax
import jax.numpy as jnp
from jax import lax
from jax.experimental import pallas as pl
from jax.experimental.pallas import tpu as pltpu
from jax.experimental.pallas import tpu_sc as plsc

B, F, V, D = 4096, 26, 2600000, 16
FD = F * D
RPS = 128 // D
SR = V // RPS

NC, NS = 2, 16
NW = NC * NS
TOT = B * F
PER_W = TOT // NW
CH = 128
NCH = PER_W // CH

KW = 8
WIN_W = KW * 128
NCOL = V // 128
NWIN = NCOL // KW
WPW = (NWIN + NW - 1) // NW
TAIL = V - NCOL * 128
TAILC = NCOL * 128

SEL_CHUNK = 8192


@functools.lru_cache(maxsize=1)
def _get_sc_transpose():
    mesh = plsc.VectorSubcoreMesh(core_axis_name="c", subcore_axis_name="s")

    @functools.partial(
        pl.kernel,
        out_type=jax.ShapeDtypeStruct((SR, 128), jnp.float32),
        mesh=mesh,
        compiler_params=pltpu.CompilerParams(needs_layout_passes=False),
        scratch_types=[
            pltpu.VMEM((16, WIN_W), jnp.float32),
            pltpu.VMEM((16, WIN_W), jnp.float32),
            pltpu.VMEM((128, 128), jnp.float32),
            pltpu.VMEM((128, 128), jnp.float32),
            pltpu.VMEM((16, TAIL), jnp.float32),
            pltpu.VMEM((RPS, 128), jnp.float32),
            pltpu.SemaphoreType.DMA,
            pltpu.SemaphoreType.DMA,
            pltpu.SemaphoreType.DMA,
            pltpu.SemaphoreType.DMA,
        ],
    )
    def _sc_transpose(tblt, out, in0, in1, ob0, ob1, tbuf, obt,
                      isem0, isem1, osem0, osem1):
        wid = lax.axis_index("s") * NC + lax.axis_index("c")
        iota = lax.iota(jnp.int32, 16)
        ins = (in0, in1)
        obs = (ob0, ob1)
        isems = (isem0, isem1)
        osems = (osem0, osem1)

        def win_id(k):
            return k * NW + wid

        def start_in(k, p):
            @pl.when(win_id(k) < NWIN)
            def _():
                off = pl.multiple_of(win_id(k) * WIN_W, WIN_W)
                pltpu.async_copy(tblt.at[:, pl.ds(off, WIN_W)], ins[p],
                                 isems[p])

        def permute(src, dst):
            def rows(jj, carry):
                base = 128 * jj
                for jm in range(16):
                    for a in range(RPS):
                        col = jnp.full((16,), base + (8 * jm + a),
                                       jnp.int32)
                        vals = plsc.load_gather(src, [iota, col])
                        dst[16 * jj + jm, pl.ds(16 * a, 16)] = vals
                return carry
            lax.fori_loop(0, KW, rows, 0)

        start_in(0, 0)

        def step(k, carry):
            def body(p):
                @pl.when((k >= 2) & (win_id(k - 2) < NWIN))
                def _():
                    pltpu.make_async_copy(
                        obs[p],
                        out.at[pl.ds(pl.multiple_of(
                            win_id(k - 2) * KW * 16, KW * 16), 128)],
                        osems[p]).wait()

                @pl.when(win_id(k) < NWIN)
                def _():
                    pltpu.make_async_copy(
                        tblt.at[:, pl.ds(pl.multiple_of(
                            win_id(k) * WIN_W, WIN_W), WIN_W)],
                        ins[p], isems[p]).wait()

                start_in(k + 1, 1 - p)

                @pl.when(win_id(k) < NWIN)
                def _():
                    permute(ins[p], obs[p])
                    off = pl.multiple_of(win_id(k) * KW * 16, KW * 16)
                    pltpu.async_copy(obs[p], out.at[pl.ds(off, 128)],
                                     osems[p])

            @pl.when(k % 2 == 0)
            def _():
                body(0)

            @pl.when(k % 2 == 1)
            def _():
                body(1)

            return carry

        lax.fori_loop(0, WPW, step, 0)

        for k in (WPW - 2, WPW - 1):
            p = k % 2

            @pl.when(win_id(k) < NWIN)
            def _():
                pltpu.make_async_copy(
                    obs[p],
                    out.at[pl.ds(pl.multiple_of(
                        win_id(k) * KW * 16, KW * 16), 128)],
                    osems[p]).wait()

        @pl.when(wid == 0)
        def _():
            pltpu.sync_copy(tblt.at[:, pl.ds(TAILC, TAIL)], tbuf)

            def trow(j, carry):
                for a in range(RPS):
                    col = jnp.full((16,), RPS * j + a, jnp.int32)
                    vals = plsc.load_gather(tbuf, [iota, col])
                    obt[j, pl.ds(16 * a, 16)] = vals
                return carry

            lax.fori_loop(0, RPS, trow, 0)
            pltpu.sync_copy(obt, out.at[pl.ds(SR - RPS, RPS)])

    return _sc_transpose


@functools.lru_cache(maxsize=1)
def _get_sc_super_gather():
    mesh = plsc.VectorSubcoreMesh(core_axis_name="c", subcore_axis_name="s")

    @functools.partial(
        pl.kernel,
        out_type=jax.ShapeDtypeStruct((TOT, 128), jnp.float32),
        mesh=mesh,
        scratch_types=[
            pltpu.VMEM((NCH, CH), jnp.int32),
            pltpu.VMEM((CH, 128), jnp.float32),
            pltpu.VMEM((CH, 128), jnp.float32),
            pltpu.SemaphoreType.DMA,
            pltpu.SemaphoreType.DMA,
        ],
    )
    def _sc_super_gather(xs_hbm, tbl8, emb_out, idx_v, buf0, buf1, sem0,
                         sem1):
        wid = lax.axis_index("s") * NC + lax.axis_index("c")
        base = wid * PER_W

        pltpu.sync_copy(xs_hbm.at[wid], idx_v)
        pltpu.async_copy(tbl8.at[idx_v.at[0]], buf0, sem0)

        def chunk(i, carry):
            bufs = ((buf0, sem0), (buf1, sem1))

            def sstep(cur, nxt):
                buf_c, sem_c = bufs[cur]
                buf_n, sem_n = bufs[nxt]
                pltpu.make_async_copy(tbl8.at[idx_v.at[i]], buf_c,
                                      sem_c).wait()

                @pl.when(i + 1 < NCH)
                def _():
                    pltpu.async_copy(tbl8.at[idx_v.at[i + 1]], buf_n, sem_n)

                pltpu.sync_copy(buf_c, emb_out.at[pl.ds(base + i * CH, CH)])

            @pl.when(i % 2 == 0)
            def _():
                sstep(0, 1)

            @pl.when(i % 2 == 1)
            def _():
                sstep(1, 0)

            return carry

        lax.fori_loop(0, NCH, chunk, 0)

    return _sc_super_gather


@functools.lru_cache(maxsize=1)
def _get_sc_o1_gather():
    mesh = plsc.VectorSubcoreMesh(core_axis_name="c", subcore_axis_name="s")

    @functools.partial(
        pl.kernel,
        out_type=jax.ShapeDtypeStruct((TOT,), jnp.float32),
        mesh=mesh,
        compiler_params=pltpu.CompilerParams(use_tc_tiling_on_sc=False),
        scratch_types=[
            pltpu.VMEM((NCH, CH), jnp.int32),
            pltpu.VMEM((PER_W,), jnp.float32),
            pltpu.SemaphoreType.DMA,
        ],
    )
    def _sc_o1_gather(x_hbm, o1_tbl, o1_out, idx_v, o1_v, sem):
        wid = lax.axis_index("s") * NC + lax.axis_index("c")
        base = wid * PER_W

        pltpu.sync_copy(x_hbm.at[wid], idx_v)

        def chunk(i, carry):
            pltpu.async_copy(o1_tbl.at[idx_v.at[i]],
                             o1_v.at[pl.ds(i * CH, CH)], sem).wait()
            return carry

        lax.fori_loop(0, NCH, chunk, 0)

        pltpu.sync_copy(o1_v, o1_out.at[pl.ds(base, PER_W)])

    return _sc_o1_gather


def _select_body(emb8_ref, sel_ref, out_ref):
    emb8 = emb8_ref[...]
    sel = sel_ref[...]
    acc = jnp.zeros((SEL_CHUNK, D), jnp.float32)
    for k in range(RPS):
        acc = acc + jnp.where(sel == k, emb8[:, k * D:(k + 1) * D], 0.0)
    out_ref[...] = acc


_select_call = pl.pallas_call(
    _select_body,
    grid=(TOT // SEL_CHUNK,),
    in_specs=[
        pl.BlockSpec((SEL_CHUNK, 128), lambda i: (i, 0)),
        pl.BlockSpec((SEL_CHUNK, D), lambda i: (i, 0)),
    ],
    out_specs=pl.BlockSpec((SEL_CHUNK, D), lambda i: (i, 0)),
    out_shape=jax.ShapeDtypeStruct((TOT, D), jnp.float32),
)


def _tc_body(emb_ref, o1_ref, w1_ref, b1_ref, g1_ref, be1_ref,
             w2_ref, b2_ref, g2_ref, be2_ref, w3_ref, b3_ref,
             w4_ref, b4_ref, out_ref):
    emb = emb_ref[...]
    o1 = jnp.sum(o1_ref[...], axis=1, keepdims=True)

    ki = lax.broadcasted_iota(jnp.int32, (FD, D), 0)
    di = lax.broadcasted_iota(jnp.int32, (FD, D), 1)
    S = (ki % D == di).astype(jnp.float32)
    sums = jnp.dot(emb, S, preferred_element_type=jnp.float32)
    sq_of_sum = jnp.sum(sums * sums, axis=1, keepdims=True)
    sum_of_sq = jnp.sum(emb * emb, axis=1, keepdims=True)
    o2 = 0.5 * (sq_of_sum - sum_of_sq)

    def bn_relu(h, g, be):
        m = jnp.mean(h, axis=0, keepdims=True)
        v = jnp.mean((h - m) * (h - m), axis=0, keepdims=True)
        return jnp.maximum((h - m) / jnp.sqrt(v + 1e-5) * g + be, 0.0)

    h = jnp.dot(emb, w1_ref[...], preferred_element_type=jnp.float32)
    h = bn_relu(h + b1_ref[...], g1_ref[...], be1_ref[...])
    h = jnp.dot(h, w2_ref[...], preferred_element_type=jnp.float32)
    h = bn_relu(h + b2_ref[...], g2_ref[...], be2_ref[...])
    h = jnp.dot(h, w3_ref[...], preferred_element_type=jnp.float32) + b3_ref[...]
    dnn = jnp.dot(h, w4_ref[...], preferred_element_type=jnp.float32) + b4_ref[...]

    out_ref[...] = o1 + o2 + dnn


_tc_call = pl.pallas_call(
    _tc_body,
    out_shape=jax.ShapeDtypeStruct((B, 1), jnp.float32),
)


def kernel(x, cat_embed, o1_fc, W1, b1, g1, be1, W2, b2, g2, be2, W3, b3,
           W4, b4):
    xi = x.astype(jnp.int32)
    xs3d = (xi >> 3).reshape(NW, NCH, CH)
    x3d = xi.reshape(NW, NCH, CH)
    sel16 = jnp.broadcast_to((xi & 7).reshape(TOT, 1), (TOT, D))

    cat8 = _get_sc_transpose()(cat_embed.T)
    emb8 = _get_sc_super_gather()(xs3d, cat8)
    o1_flat = _get_sc_o1_gather()(x3d, o1_fc.reshape(V))
    emb_flat = _select_call(emb8, sel16)

    emb = emb_flat.reshape(B, FD)
    o1v = o1_flat.reshape(B, F)
    return _tc_call(emb, o1v, W1.T, b1.reshape(1, -1), g1.reshape(1, -1),
                    be1.reshape(1, -1), W2.T, b2.reshape(1, -1),
                    g2.reshape(1, -1), be2.reshape(1, -1), W3.T,
                    b3.reshape(1, -1), W4.T, b4.reshape(1, -1))


# speedup vs baseline: 1.1520x; 1.1520x over previous
import functools

import jax
import jax.numpy as jnp
from jax import lax
from jax.experimental import pallas as pl
from jax.experimental.pallas import tpu as pltpu
from jax.experimental.pallas import tpu_sc as plsc

B, F, V, D = 4096, 26, 2600000, 16
FD = F * D
MLP = [512, 256, 128]

NC, NS = 2, 16
NW = NC * NS
TOT = B * F
PER_W = TOT // NW
CH = 128
NCH = PER_W // CH


@functools.lru_cache(maxsize=1)
def _get_sc_gather():
    mesh = plsc.VectorSubcoreMesh(core_axis_name="c", subcore_axis_name="s")

    @functools.partial(
        pl.kernel,
        out_type=[
            jax.ShapeDtypeStruct((NW, PER_W, D), jnp.float32),
            jax.ShapeDtypeStruct((TOT,), jnp.float32),
        ],
        mesh=mesh,
        compiler_params=pltpu.CompilerParams(use_tc_tiling_on_sc=False),
        scratch_types=[
            pltpu.VMEM((NCH, CH), jnp.int32),
            pltpu.VMEM((PER_W, D), jnp.float32),
            pltpu.VMEM((PER_W,), jnp.float32),
            pltpu.SemaphoreType.DMA,
            pltpu.SemaphoreType.DMA,
        ],
    )
    def _sc_gather(x_hbm, emb_tbl, o1_tbl, emb_out, o1_out, idx_v, rows_v,
                   o1_v, sem_e, sem_o):
        wid = lax.axis_index("s") * NC + lax.axis_index("c")
        base = wid * PER_W

        pltpu.sync_copy(x_hbm.at[wid], idx_v)

        def chunk(i, carry):
            idx_row = idx_v.at[i]
            ce = pltpu.async_copy(emb_tbl.at[idx_row],
                                  rows_v.at[pl.ds(i * CH, CH)], sem_e)
            co = pltpu.async_copy(o1_tbl.at[idx_row],
                                  o1_v.at[pl.ds(i * CH, CH)], sem_o)
            ce.wait()
            co.wait()
            return carry

        lax.fori_loop(0, NCH, chunk, 0)

        pltpu.sync_copy(rows_v, emb_out.at[wid])
        pltpu.sync_copy(o1_v, o1_out.at[pl.ds(base, PER_W)])

    return _sc_gather


def _tc_body(emb_ref, o1_ref, w1_ref, b1_ref, g1_ref, be1_ref,
             w2_ref, b2_ref, g2_ref, be2_ref, w3_ref, b3_ref,
             w4_ref, b4_ref, out_ref):
    emb = emb_ref[...]
    o1 = jnp.sum(o1_ref[...], axis=1, keepdims=True)

    ki = lax.broadcasted_iota(jnp.int32, (FD, D), 0)
    di = lax.broadcasted_iota(jnp.int32, (FD, D), 1)
    S = (ki % D == di).astype(jnp.float32)
    sums = jnp.dot(emb, S, preferred_element_type=jnp.float32)
    sq_of_sum = jnp.sum(sums * sums, axis=1, keepdims=True)
    sum_of_sq = jnp.sum(emb * emb, axis=1, keepdims=True)
    o2 = 0.5 * (sq_of_sum - sum_of_sq)

    def bn_relu(h, g, be):
        m = jnp.mean(h, axis=0, keepdims=True)
        v = jnp.mean((h - m) * (h - m), axis=0, keepdims=True)
        return jnp.maximum((h - m) / jnp.sqrt(v + 1e-5) * g + be, 0.0)

    h = jnp.dot(emb, w1_ref[...], preferred_element_type=jnp.float32)
    h = bn_relu(h + b1_ref[...], g1_ref[...], be1_ref[...])
    h = jnp.dot(h, w2_ref[...], preferred_element_type=jnp.float32)
    h = bn_relu(h + b2_ref[...], g2_ref[...], be2_ref[...])
    h = jnp.dot(h, w3_ref[...], preferred_element_type=jnp.float32) + b3_ref[...]
    dnn = jnp.dot(h, w4_ref[...], preferred_element_type=jnp.float32) + b4_ref[...]

    out_ref[...] = o1 + o2 + dnn


_tc_call = pl.pallas_call(
    _tc_body,
    out_shape=jax.ShapeDtypeStruct((B, 1), jnp.float32),
)


def kernel(x, cat_embed, o1_fc, W1, b1, g1, be1, W2, b2, g2, be2, W3, b3,
           W4, b4):
    x3d = x.astype(jnp.int32).reshape(NW, NCH, CH)
    emb3, o1_flat = _get_sc_gather()(x3d, cat_embed, o1_fc[:, 0])
    emb = emb3.reshape(B, FD)
    o1v = o1_flat.reshape(B, F)
    return _tc_call(emb, o1v, W1.T, b1.reshape(1, -1), g1.reshape(1, -1),
                    be1.reshape(1, -1), W2.T, b2.reshape(1, -1),
                    g2.reshape(1, -1), be2.reshape(1, -1), W3.T,
                    b3.reshape(1, -1), W4.T, b4.reshape(1, -1))
